# Initial kernel scaffold; baseline (speedup 1.0000x reference)
import jax
import jax.numpy as jnp
from jax.experimental import pallas as pl


def kernel(nodes, loc, edges, vel, edge_attr, charges, params):
    raise NotImplementedError("write your pallas kernel here")


# speedup vs baseline: 24.0322x; 24.0322x over previous
import functools

import jax
import jax.numpy as jnp
from jax import lax
from jax.experimental import pallas as pl
from jax.experimental.pallas import tpu as pltpu
from jax.experimental.pallas import tpu_sc as plsc

N = 50000
E = 200000
B = 10000
NPART = 5
HID = 64
EPS = 1e-6

NW = 32
N_PAD = 50176
ROWS_PER_TILE = N_PAD // 16
ZROWS = 784
E_PAD = 200704
EDGES_PER_W = E_PAD // NW
EBATCH = 128
NBATCH = EDGES_PER_W // EBATCH

_mesh = plsc.VectorSubcoreMesh(core_axis_name="c", subcore_axis_name="s")


@functools.partial(
    pl.kernel,
    mesh=_mesh,
    out_type=jax.ShapeDtypeStruct((2, N_PAD, 16), jnp.float32),
    scratch_types=[
        pltpu.VMEM((ZROWS, 16), jnp.float32),
        pltpu.VMEM((EBATCH,), jnp.int32),
        pltpu.VMEM((EBATCH,), jnp.int32),
        pltpu.VMEM((EBATCH, 16), jnp.float32),
        pltpu.SemaphoreType.DMA,
        pltpu.VMEM_SHARED((N_PAD, 16), jnp.float32),
    ],
    compiler_params=pltpu.CompilerParams(use_tc_tiling_on_sc=False),
)
def _sc_seg0(x_hbm, src_hbm, dst_hbm, zb_hbm, out_hbm,
             zbuf, idx_s, idx_d, rows, sem, acc):
    c = lax.axis_index("c")
    s = lax.axis_index("s")
    pltpu.sync_copy(zb_hbm, zbuf)
    for j in range(4):
        pltpu.sync_copy(zbuf, acc.at[pl.ds(s * ROWS_PER_TILE + j * ZROWS, ZROWS)])
    plsc.subcore_barrier()
    base = (c * 16 + s) * EDGES_PER_W

    def body(b, carry):
        off = base + b * EBATCH
        pltpu.sync_copy(src_hbm.at[pl.ds(off, EBATCH)], idx_s)
        pltpu.sync_copy(dst_hbm.at[pl.ds(off, EBATCH)], idx_d)
        pltpu.async_copy(x_hbm.at[idx_s], rows, sem).wait()
        pltpu.sync_copy(rows, acc.at[idx_d], add=True)
        return carry

    lax.fori_loop(0, NBATCH, body, 0)
    plsc.subcore_barrier()
    for j in range(4):
        r0 = s * ROWS_PER_TILE + j * ZROWS
        pltpu.sync_copy(acc.at[pl.ds(r0, ZROWS)], out_hbm.at[c].at[pl.ds(r0, ZROWS)])


@functools.partial(
    pl.kernel,
    mesh=_mesh,
    out_type=jax.ShapeDtypeStruct((2, 6, N_PAD, 32), jnp.float32),
    scratch_types=[
        pltpu.VMEM((ZROWS, 32), jnp.float32),
        pltpu.VMEM((EBATCH,), jnp.int32),
        pltpu.VMEM((EBATCH,), jnp.int32),
        pltpu.VMEM((EBATCH, 32), jnp.float32),
        pltpu.SemaphoreType.DMA,
        pltpu.VMEM_SHARED((N_PAD, 32), jnp.float32),
    ],
    compiler_params=pltpu.CompilerParams(use_tc_tiling_on_sc=False),
)
def _sc_segh(x_hbm, src_hbm, dst_hbm, zb_hbm, out_hbm,
             zbuf, idx_s, idx_d, rows, sem, acc):
    c = lax.axis_index("c")
    s = lax.axis_index("s")
    pltpu.sync_copy(zb_hbm, zbuf)
    base = (c * 16 + s) * EDGES_PER_W
    for k in range(6):
        for j in range(4):
            pltpu.sync_copy(zbuf, acc.at[pl.ds(s * ROWS_PER_TILE + j * ZROWS, ZROWS)])
        plsc.subcore_barrier()

        def body(b, carry):
            off = base + b * EBATCH
            pltpu.sync_copy(src_hbm.at[pl.ds(off, EBATCH)], idx_s)
            pltpu.sync_copy(dst_hbm.at[pl.ds(off, EBATCH)], idx_d)
            pltpu.async_copy(x_hbm.at[k].at[idx_s], rows, sem).wait()
            pltpu.sync_copy(rows, acc.at[idx_d], add=True)
            return carry

        lax.fori_loop(0, NBATCH, body, 0)
        plsc.subcore_barrier()
        for j in range(4):
            r0 = s * ROWS_PER_TILE + j * ZROWS
            pltpu.sync_copy(acc.at[pl.ds(r0, ZROWS)],
                            out_hbm.at[c].at[k].at[pl.ds(r0, ZROWS)])
        plsc.subcore_barrier()


def _tc_prep_body(loc_ref, vel_ref, out_ref):
    l = loc_ref[...]
    v = vel_ref[...]
    rows = l.shape[0]
    g = l.reshape(rows // NPART, NPART, 3)
    canon = (g - jnp.mean(g, axis=1, keepdims=True)).reshape(rows, 3)
    ones = jnp.ones((rows, 1), jnp.float32)
    zeros = jnp.zeros((rows, 9), jnp.float32)
    out_ref[...] = jnp.concatenate(
        [canon[:, 0:1], v[:, 0:1], canon[:, 1:2], v[:, 1:2],
         canon[:, 2:3], v[:, 2:3], ones, zeros], axis=1)


def _tc_prep(loc, vel):
    bb = 2000
    grid = (N // bb,)
    return pl.pallas_call(
        _tc_prep_body,
        grid=grid,
        in_specs=[pl.BlockSpec((bb, 3), lambda i: (i, 0)),
                  pl.BlockSpec((bb, 3), lambda i: (i, 0))],
        out_specs=pl.BlockSpec((bb, 16), lambda i: (i, 0)),
        out_shape=jax.ShapeDtypeStruct((N, 16), jnp.float32),
    )(loc, vel)


def _tc_layer0_body(x_ref, ps_ref, idw_ref, idb_ref, plw_ref, plb_ref,
                    dirw_ref, y_ref):
    x = x_ref[...]
    ps = ps_ref[0] + ps_ref[1]
    cnt = jnp.maximum(ps[:, 6:7], 1.0)
    idW = idw_ref[...]
    plW = plw_ref[...]
    dirW = dirw_ref[...]
    idb = idb_ref[...]
    plb = plb_ref[...]
    pre = []
    for k in range(3):
        xk = x[:, 2 * k:2 * k + 2]
        pk = ps[:, 2 * k:2 * k + 2] / cnt
        ik = jnp.dot(xk, idW.T, preferred_element_type=jnp.float32) + idb
        pl_k = jnp.dot(pk, plW.T, preferred_element_type=jnp.float32) + plb
        pre.append(ik + pl_k)
    d = [jnp.dot(pre[k], dirW.T, preferred_element_type=jnp.float32)
         for k in range(3)]
    dot = sum(pre[k] * d[k] for k in range(3))
    d2 = sum(d[k] * d[k] for k in range(3))
    coef = jnp.where(dot >= 0.0, 0.0, dot / (d2 + EPS))
    for k in range(3):
        yk = pre[k] - coef * d[k]
        y_ref[2 * k] = yk[:, 0:32]
        y_ref[2 * k + 1] = yk[:, 32:64]


def _tc_layer0(x0, ps0, idW, idb, plW, plb, dirW):
    bn = 1000
    grid = (N // bn,)
    return pl.pallas_call(
        _tc_layer0_body,
        grid=grid,
        in_specs=[
            pl.BlockSpec((bn, 16), lambda i: (i, 0)),
            pl.BlockSpec((2, bn, 16), lambda i: (0, i, 0)),
            pl.BlockSpec((HID, 2), lambda i: (0, 0)),
            pl.BlockSpec((1, HID), lambda i: (0, 0)),
            pl.BlockSpec((HID, 2), lambda i: (0, 0)),
            pl.BlockSpec((1, HID), lambda i: (0, 0)),
            pl.BlockSpec((HID, HID), lambda i: (0, 0)),
        ],
        out_specs=pl.BlockSpec((6, bn, 32), lambda i: (0, i, 0)),
        out_shape=jax.ShapeDtypeStruct((6, N, 32), jnp.float32),
    )(x0, ps0, idW, idb, plW, plb, dirW)


def _tc_layerh_body(x_ref, ps_ref, c_ref, idw_ref, idb_ref, plw_ref,
                    plb_ref, dirw_ref, y_ref):
    cnt0 = c_ref[0] + c_ref[1]
    cnt = jnp.maximum(cnt0[:, 6:7], 1.0)
    idW = idw_ref[...]
    plW = plw_ref[...]
    dirW = dirw_ref[...]
    idb = idb_ref[...]
    plb = plb_ref[...]
    xs = []
    pre = []
    for k in range(3):
        xk = jnp.concatenate([x_ref[2 * k], x_ref[2 * k + 1]], axis=1)
        sk = jnp.concatenate([ps_ref[0, 2 * k] + ps_ref[1, 2 * k],
                              ps_ref[0, 2 * k + 1] + ps_ref[1, 2 * k + 1]],
                             axis=1)
        pk = sk / cnt
        ik = jnp.dot(xk, idW.T, preferred_element_type=jnp.float32) + idb
        pl_k = jnp.dot(pk, plW.T, preferred_element_type=jnp.float32) + plb
        xs.append(xk)
        pre.append(ik + pl_k)
    d = [jnp.dot(pre[k], dirW.T, preferred_element_type=jnp.float32)
         for k in range(3)]
    dot = sum(pre[k] * d[k] for k in range(3))
    d2 = sum(d[k] * d[k] for k in range(3))
    coef = jnp.where(dot >= 0.0, 0.0, dot / (d2 + EPS))
    for k in range(3):
        yk = pre[k] - coef * d[k] + xs[k]
        y_ref[2 * k] = yk[:, 0:32]
        y_ref[2 * k + 1] = yk[:, 32:64]


def _tc_layerh(x, ps, ps0, idW, idb, plW, plb, dirW):
    bn = 1000
    grid = (N // bn,)
    return pl.pallas_call(
        _tc_layerh_body,
        grid=grid,
        in_specs=[
            pl.BlockSpec((6, bn, 32), lambda i: (0, i, 0)),
            pl.BlockSpec((2, 6, bn, 32), lambda i: (0, 0, i, 0)),
            pl.BlockSpec((2, bn, 16), lambda i: (0, i, 0)),
            pl.BlockSpec((HID, HID), lambda i: (0, 0)),
            pl.BlockSpec((1, HID), lambda i: (0, 0)),
            pl.BlockSpec((HID, HID), lambda i: (0, 0)),
            pl.BlockSpec((1, HID), lambda i: (0, 0)),
            pl.BlockSpec((HID, HID), lambda i: (0, 0)),
        ],
        out_specs=pl.BlockSpec((6, bn, 32), lambda i: (0, i, 0)),
        out_shape=jax.ShapeDtypeStruct((6, N, 32), jnp.float32),
    )(x, ps, ps0, idW, idb, plW, plb, dirW)


def _tc_final_body(x_ref, ow_ref, ob_ref, out_ref):
    oW = ow_ref[...]
    ob = ob_ref[...]
    outs = []
    for k in range(3):
        xk = jnp.concatenate([x_ref[2 * k], x_ref[2 * k + 1]], axis=1)
        rows = xk.shape[0]
        g = jnp.mean(xk.reshape(rows // NPART, NPART, HID), axis=1)
        outs.append(jnp.dot(g, oW.T, preferred_element_type=jnp.float32) + ob)
    out_ref[...] = jnp.stack(outs, axis=1)


def _tc_final(x, oW, ob):
    bb = 400
    grid = (B // bb,)
    return pl.pallas_call(
        _tc_final_body,
        grid=grid,
        in_specs=[
            pl.BlockSpec((6, bb * NPART, 32), lambda i: (0, i, 0)),
            pl.BlockSpec((4, HID), lambda i: (0, 0)),
            pl.BlockSpec((1, 4), lambda i: (0, 0)),
        ],
        out_specs=pl.BlockSpec((bb, 3, 4), lambda i: (i, 0, 0)),
        out_shape=jax.ShapeDtypeStruct((B, 3, 4), jnp.float32),
    )(x, oW, ob)


def kernel(nodes, loc, edges, vel, edge_attr, charges, params):
    src = edges[0]
    dst = edges[1]
    srcp = jnp.concatenate([src, jnp.zeros((E_PAD - E,), jnp.int32)])
    dstp = jnp.concatenate([dst, jnp.full((E_PAD - E,), N, jnp.int32)])
    zb16 = jnp.zeros((ZROWS, 16), jnp.float32)
    zb32 = jnp.zeros((ZROWS, 32), jnp.float32)

    x0 = _tc_prep(loc, vel)
    ps0 = _sc_seg0(x0, srcp, dstp, zb16)
    x = _tc_layer0(x0, ps0,
                   params["id_W0"], params["id_b0"].reshape(1, HID),
                   params["pool_W0"], params["pool_b0"].reshape(1, HID),
                   params["dir_W0"])
    for i in range(1, 4):
        ps = _sc_segh(x, srcp, dstp, zb32)
        x = _tc_layerh(x, ps, ps0,
                       params["id_W%d" % i], params["id_b%d" % i].reshape(1, HID),
                       params["pool_W%d" % i], params["pool_b%d" % i].reshape(1, HID),
                       params["dir_W%d" % i])
    o = _tc_final(x, params["out_W"], params["out_b"].reshape(1, 4))
    o = jnp.swapaxes(o, 1, 2)
    return o[:, :3, :], o[:, 3:, :]


# speedup vs baseline: 34.3662x; 1.4300x over previous
import functools

import jax
import jax.numpy as jnp
from jax import lax
from jax.experimental import pallas as pl
from jax.experimental.pallas import tpu as pltpu
from jax.experimental.pallas import tpu_sc as plsc

N = 50000
E = 200000
B = 10000
NPART = 5
HID = 64
EPS = 1e-6

NW = 32
N_PAD = 50176
ROWS_PER_TILE = N_PAD // 16
WROWS = 784
ZROWS = 112
E_PAD = 200704
EDGES_PER_W = E_PAD // NW
EBATCH = 128
NBATCH = EDGES_PER_W // EBATCH

_mesh = plsc.VectorSubcoreMesh(core_axis_name="c", subcore_axis_name="s")


def _seg_pass(xk, acc, idxs_all, idxd_all, rows_a, rows_b, sem_a, sem_b):
    pltpu.async_copy(xk.at[idxs_all.at[0]], rows_a, sem_a)

    def body(i, carry):
        b0 = 2 * i
        pltpu.async_copy(xk.at[idxs_all.at[b0 + 1]], rows_b, sem_b)
        pltpu.make_async_copy(xk.at[idxs_all.at[b0]], rows_a, sem_a).wait()
        pltpu.sync_copy(rows_a, acc.at[idxd_all.at[b0]], add=True)
        pltpu.async_copy(xk.at[idxs_all.at[b0 + 2]], rows_a, sem_a)
        pltpu.make_async_copy(xk.at[idxs_all.at[b0 + 1]], rows_b, sem_b).wait()
        pltpu.sync_copy(rows_b, acc.at[idxd_all.at[b0 + 1]], add=True)
        return carry

    lax.fori_loop(0, (NBATCH - 1) // 2, body, 0)
    pltpu.make_async_copy(xk.at[idxs_all.at[NBATCH - 1]], rows_a, sem_a).wait()
    pltpu.sync_copy(rows_a, acc.at[idxd_all.at[NBATCH - 1]], add=True)


@functools.partial(
    pl.kernel,
    mesh=_mesh,
    out_type=jax.ShapeDtypeStruct((2, N_PAD, 16), jnp.float32),
    scratch_types=[
        pltpu.VMEM((ZROWS, 16), jnp.float32),
        pltpu.VMEM((NBATCH, EBATCH), jnp.int32),
        pltpu.VMEM((NBATCH, EBATCH), jnp.int32),
        pltpu.VMEM((EBATCH, 16), jnp.float32),
        pltpu.VMEM((EBATCH, 16), jnp.float32),
        pltpu.SemaphoreType.DMA,
        pltpu.SemaphoreType.DMA,
        pltpu.VMEM_SHARED((N_PAD, 16), jnp.float32),
    ],
    compiler_params=pltpu.CompilerParams(use_tc_tiling_on_sc=False),
)
def _sc_seg0(x_hbm, src_hbm, dst_hbm, zb_hbm, out_hbm,
             zbuf, idxs_all, idxd_all, rows_a, rows_b, sem_a, sem_b, acc):
    c = lax.axis_index("c")
    s = lax.axis_index("s")
    w = c * 16 + s
    pltpu.sync_copy(zb_hbm, zbuf)
    pltpu.sync_copy(src_hbm.at[pl.ds(w * NBATCH, NBATCH)], idxs_all)
    pltpu.sync_copy(dst_hbm.at[pl.ds(w * NBATCH, NBATCH)], idxd_all)
    for j in range(28):
        pltpu.sync_copy(zbuf, acc.at[pl.ds(s * ROWS_PER_TILE + j * ZROWS, ZROWS)])
    plsc.subcore_barrier()
    _seg_pass(x_hbm, acc, idxs_all, idxd_all, rows_a, rows_b, sem_a, sem_b)
    plsc.subcore_barrier()
    for j in range(4):
        r0 = s * ROWS_PER_TILE + j * WROWS
        pltpu.sync_copy(acc.at[pl.ds(r0, WROWS)], out_hbm.at[c].at[pl.ds(r0, WROWS)])


@functools.partial(
    pl.kernel,
    mesh=_mesh,
    out_type=jax.ShapeDtypeStruct((2, 6, N_PAD, 32), jnp.float32),
    scratch_types=[
        pltpu.VMEM((ZROWS, 32), jnp.float32),
        pltpu.VMEM((NBATCH, EBATCH), jnp.int32),
        pltpu.VMEM((NBATCH, EBATCH), jnp.int32),
        pltpu.VMEM((EBATCH, 32), jnp.float32),
        pltpu.VMEM((EBATCH, 32), jnp.float32),
        pltpu.SemaphoreType.DMA,
        pltpu.SemaphoreType.DMA,
        pltpu.VMEM_SHARED((N_PAD, 32), jnp.float32),
    ],
    compiler_params=pltpu.CompilerParams(use_tc_tiling_on_sc=False),
)
def _sc_segh(x_hbm, src_hbm, dst_hbm, zb_hbm, out_hbm,
             zbuf, idxs_all, idxd_all, rows_a, rows_b, sem_a, sem_b, acc):
    c = lax.axis_index("c")
    s = lax.axis_index("s")
    w = c * 16 + s
    pltpu.sync_copy(zb_hbm, zbuf)
    pltpu.sync_copy(src_hbm.at[pl.ds(w * NBATCH, NBATCH)], idxs_all)
    pltpu.sync_copy(dst_hbm.at[pl.ds(w * NBATCH, NBATCH)], idxd_all)
    for k in range(6):
        for j in range(28):
            pltpu.sync_copy(zbuf, acc.at[pl.ds(s * ROWS_PER_TILE + j * ZROWS, ZROWS)])
        plsc.subcore_barrier()
        _seg_pass(x_hbm.at[k], acc, idxs_all, idxd_all,
                  rows_a, rows_b, sem_a, sem_b)
        plsc.subcore_barrier()
        for j in range(4):
            r0 = s * ROWS_PER_TILE + j * WROWS
            pltpu.sync_copy(acc.at[pl.ds(r0, WROWS)],
                            out_hbm.at[c].at[k].at[pl.ds(r0, WROWS)])
        plsc.subcore_barrier()


def _tc_prep_body(loc_ref, vel_ref, out_ref):
    l = loc_ref[...]
    v = vel_ref[...]
    rows = l.shape[0]
    g = l.reshape(rows // NPART, NPART, 3)
    canon = (g - jnp.mean(g, axis=1, keepdims=True)).reshape(rows, 3)
    ones = jnp.ones((rows, 1), jnp.float32)
    zeros = jnp.zeros((rows, 9), jnp.float32)
    out_ref[...] = jnp.concatenate(
        [canon[:, 0:1], v[:, 0:1], canon[:, 1:2], v[:, 1:2],
         canon[:, 2:3], v[:, 2:3], ones, zeros], axis=1)


def _tc_prep(loc, vel):
    bb = 2000
    grid = (N // bb,)
    return pl.pallas_call(
        _tc_prep_body,
        grid=grid,
        in_specs=[pl.BlockSpec((bb, 3), lambda i: (i, 0)),
                  pl.BlockSpec((bb, 3), lambda i: (i, 0))],
        out_specs=pl.BlockSpec((bb, 16), lambda i: (i, 0)),
        out_shape=jax.ShapeDtypeStruct((N, 16), jnp.float32),
    )(loc, vel)


def _tc_layer0_body(x_ref, ps_ref, idw_ref, idb_ref, plw_ref, plb_ref,
                    dirw_ref, y_ref):
    x = x_ref[...]
    ps = ps_ref[0] + ps_ref[1]
    cnt = jnp.maximum(ps[:, 6:7], 1.0)
    idW = idw_ref[...]
    plW = plw_ref[...]
    dirW = dirw_ref[...]
    idb = idb_ref[...]
    plb = plb_ref[...]
    pre = []
    for k in range(3):
        xk = x[:, 2 * k:2 * k + 2]
        pk = ps[:, 2 * k:2 * k + 2] / cnt
        ik = jnp.dot(xk, idW.T, preferred_element_type=jnp.float32) + idb
        pl_k = jnp.dot(pk, plW.T, preferred_element_type=jnp.float32) + plb
        pre.append(ik + pl_k)
    d = [jnp.dot(pre[k], dirW.T, preferred_element_type=jnp.float32)
         for k in range(3)]
    dot = sum(pre[k] * d[k] for k in range(3))
    d2 = sum(d[k] * d[k] for k in range(3))
    coef = jnp.where(dot >= 0.0, 0.0, dot / (d2 + EPS))
    for k in range(3):
        yk = pre[k] - coef * d[k]
        y_ref[2 * k] = yk[:, 0:32]
        y_ref[2 * k + 1] = yk[:, 32:64]


def _tc_layer0(x0, ps0, idW, idb, plW, plb, dirW):
    bn = 1000
    grid = (N // bn,)
    return pl.pallas_call(
        _tc_layer0_body,
        grid=grid,
        in_specs=[
            pl.BlockSpec((bn, 16), lambda i: (i, 0)),
            pl.BlockSpec((2, bn, 16), lambda i: (0, i, 0)),
            pl.BlockSpec((HID, 2), lambda i: (0, 0)),
            pl.BlockSpec((1, HID), lambda i: (0, 0)),
            pl.BlockSpec((HID, 2), lambda i: (0, 0)),
            pl.BlockSpec((1, HID), lambda i: (0, 0)),
            pl.BlockSpec((HID, HID), lambda i: (0, 0)),
        ],
        out_specs=pl.BlockSpec((6, bn, 32), lambda i: (0, i, 0)),
        out_shape=jax.ShapeDtypeStruct((6, N, 32), jnp.float32),
    )(x0, ps0, idW, idb, plW, plb, dirW)


def _tc_layerh_body(x_ref, ps_ref, c_ref, idw_ref, idb_ref, plw_ref,
                    plb_ref, dirw_ref, y_ref):
    cnt0 = c_ref[0] + c_ref[1]
    cnt = jnp.maximum(cnt0[:, 6:7], 1.0)
    idW = idw_ref[...]
    plW = plw_ref[...]
    dirW = dirw_ref[...]
    idb = idb_ref[...]
    plb = plb_ref[...]
    xs = []
    pre = []
    for k in range(3):
        xk = jnp.concatenate([x_ref[2 * k], x_ref[2 * k + 1]], axis=1)
        sk = jnp.concatenate([ps_ref[0, 2 * k] + ps_ref[1, 2 * k],
                              ps_ref[0, 2 * k + 1] + ps_ref[1, 2 * k + 1]],
                             axis=1)
        pk = sk / cnt
        ik = jnp.dot(xk, idW.T, preferred_element_type=jnp.float32) + idb
        pl_k = jnp.dot(pk, plW.T, preferred_element_type=jnp.float32) + plb
        xs.append(xk)
        pre.append(ik + pl_k)
    d = [jnp.dot(pre[k], dirW.T, preferred_element_type=jnp.float32)
         for k in range(3)]
    dot = sum(pre[k] * d[k] for k in range(3))
    d2 = sum(d[k] * d[k] for k in range(3))
    coef = jnp.where(dot >= 0.0, 0.0, dot / (d2 + EPS))
    for k in range(3):
        yk = pre[k] - coef * d[k] + xs[k]
        y_ref[2 * k] = yk[:, 0:32]
        y_ref[2 * k + 1] = yk[:, 32:64]


def _tc_layerh(x, ps, ps0, idW, idb, plW, plb, dirW):
    bn = 1000
    grid = (N // bn,)
    return pl.pallas_call(
        _tc_layerh_body,
        grid=grid,
        in_specs=[
            pl.BlockSpec((6, bn, 32), lambda i: (0, i, 0)),
            pl.BlockSpec((2, 6, bn, 32), lambda i: (0, 0, i, 0)),
            pl.BlockSpec((2, bn, 16), lambda i: (0, i, 0)),
            pl.BlockSpec((HID, HID), lambda i: (0, 0)),
            pl.BlockSpec((1, HID), lambda i: (0, 0)),
            pl.BlockSpec((HID, HID), lambda i: (0, 0)),
            pl.BlockSpec((1, HID), lambda i: (0, 0)),
            pl.BlockSpec((HID, HID), lambda i: (0, 0)),
        ],
        out_specs=pl.BlockSpec((6, bn, 32), lambda i: (0, i, 0)),
        out_shape=jax.ShapeDtypeStruct((6, N, 32), jnp.float32),
    )(x, ps, ps0, idW, idb, plW, plb, dirW)


def _tc_final_body(x_ref, ow_ref, ob_ref, out_ref):
    oW = ow_ref[...]
    ob = ob_ref[...]
    outs = []
    for k in range(3):
        xk = jnp.concatenate([x_ref[2 * k], x_ref[2 * k + 1]], axis=1)
        rows = xk.shape[0]
        g = jnp.mean(xk.reshape(rows // NPART, NPART, HID), axis=1)
        outs.append(jnp.dot(g, oW.T, preferred_element_type=jnp.float32) + ob)
    out_ref[...] = jnp.stack(outs, axis=1)


def _tc_final(x, oW, ob):
    bb = 400
    grid = (B // bb,)
    return pl.pallas_call(
        _tc_final_body,
        grid=grid,
        in_specs=[
            pl.BlockSpec((6, bb * NPART, 32), lambda i: (0, i, 0)),
            pl.BlockSpec((4, HID), lambda i: (0, 0)),
            pl.BlockSpec((1, 4), lambda i: (0, 0)),
        ],
        out_specs=pl.BlockSpec((bb, 3, 4), lambda i: (i, 0, 0)),
        out_shape=jax.ShapeDtypeStruct((B, 3, 4), jnp.float32),
    )(x, oW, ob)


def kernel(nodes, loc, edges, vel, edge_attr, charges, params):
    src = edges[0]
    dst = edges[1]
    srcp = jnp.concatenate([src, jnp.zeros((E_PAD - E,), jnp.int32)])
    dstp = jnp.concatenate([dst, jnp.full((E_PAD - E,), N, jnp.int32)])
    srcp = srcp.reshape(NW * NBATCH, EBATCH)
    dstp = dstp.reshape(NW * NBATCH, EBATCH)
    zb16 = jnp.zeros((ZROWS, 16), jnp.float32)
    zb32 = jnp.zeros((ZROWS, 32), jnp.float32)

    x0 = _tc_prep(loc, vel)
    ps0 = _sc_seg0(x0, srcp, dstp, zb16)
    x = _tc_layer0(x0, ps0,
                   params["id_W0"], params["id_b0"].reshape(1, HID),
                   params["pool_W0"], params["pool_b0"].reshape(1, HID),
                   params["dir_W0"])
    for i in range(1, 4):
        ps = _sc_segh(x, srcp, dstp, zb32)
        x = _tc_layerh(x, ps, ps0,
                       params["id_W%d" % i], params["id_b%d" % i].reshape(1, HID),
                       params["pool_W%d" % i], params["pool_b%d" % i].reshape(1, HID),
                       params["dir_W%d" % i])
    o = _tc_final(x, params["out_W"], params["out_b"].reshape(1, 4))
    o = jnp.swapaxes(o, 1, 2)
    return o[:, :3, :], o[:, 3:, :]


# speedup vs baseline: 34.5197x; 1.0045x over previous
import functools

import jax
import jax.numpy as jnp
from jax import lax
from jax.experimental import pallas as pl
from jax.experimental.pallas import tpu as pltpu
from jax.experimental.pallas import tpu_sc as plsc

N = 50000
E = 200000
B = 10000
NPART = 5
HID = 64
EPS = 1e-6

NW = 32
N_PAD = 50176
ROWS_PER_TILE = N_PAD // 16
WROWS = 784
ZROWS = 112
E_PAD = 200704
EDGES_PER_W = E_PAD // NW
EBATCH = 128
NBATCH = EDGES_PER_W // EBATCH

_mesh = plsc.VectorSubcoreMesh(core_axis_name="c", subcore_axis_name="s")


def _seg_pass(xk, acc, idxs_all, idxd_all, rows_a, rows_b, sem_a, sem_b):
    pltpu.async_copy(xk.at[idxs_all.at[0]], rows_a, sem_a)

    def body(i, carry):
        b0 = 2 * i
        pltpu.async_copy(xk.at[idxs_all.at[b0 + 1]], rows_b, sem_b)
        pltpu.make_async_copy(xk.at[idxs_all.at[b0]], rows_a, sem_a).wait()
        pltpu.sync_copy(rows_a, acc.at[idxd_all.at[b0]], add=True)
        pltpu.async_copy(xk.at[idxs_all.at[b0 + 2]], rows_a, sem_a)
        pltpu.make_async_copy(xk.at[idxs_all.at[b0 + 1]], rows_b, sem_b).wait()
        pltpu.sync_copy(rows_b, acc.at[idxd_all.at[b0 + 1]], add=True)
        return carry

    lax.fori_loop(0, (NBATCH - 1) // 2, body, 0)
    pltpu.make_async_copy(xk.at[idxs_all.at[NBATCH - 1]], rows_a, sem_a).wait()
    pltpu.sync_copy(rows_a, acc.at[idxd_all.at[NBATCH - 1]], add=True)


@functools.partial(
    pl.kernel,
    mesh=_mesh,
    out_type=jax.ShapeDtypeStruct((2, N_PAD, 16), jnp.float32),
    scratch_types=[
        pltpu.VMEM((ZROWS, 16), jnp.float32),
        pltpu.VMEM((NBATCH, EBATCH), jnp.int32),
        pltpu.VMEM((NBATCH, EBATCH), jnp.int32),
        pltpu.VMEM((EBATCH, 16), jnp.float32),
        pltpu.VMEM((EBATCH, 16), jnp.float32),
        pltpu.SemaphoreType.DMA,
        pltpu.SemaphoreType.DMA,
        pltpu.VMEM_SHARED((N_PAD, 16), jnp.float32),
    ],
    compiler_params=pltpu.CompilerParams(use_tc_tiling_on_sc=False),
)
def _sc_seg0(x_hbm, src_hbm, dst_hbm, zb_hbm, out_hbm,
             zbuf, idxs_all, idxd_all, rows_a, rows_b, sem_a, sem_b, acc):
    c = lax.axis_index("c")
    s = lax.axis_index("s")
    w = c * 16 + s
    pltpu.sync_copy(zb_hbm, zbuf)
    pltpu.sync_copy(src_hbm.at[pl.ds(w * NBATCH, NBATCH)], idxs_all)
    pltpu.sync_copy(dst_hbm.at[pl.ds(w * NBATCH, NBATCH)], idxd_all)
    for j in range(28):
        pltpu.sync_copy(zbuf, acc.at[pl.ds(s * ROWS_PER_TILE + j * ZROWS, ZROWS)])
    plsc.subcore_barrier()
    _seg_pass(x_hbm, acc, idxs_all, idxd_all, rows_a, rows_b, sem_a, sem_b)
    plsc.subcore_barrier()
    for j in range(4):
        r0 = s * ROWS_PER_TILE + j * WROWS
        pltpu.sync_copy(acc.at[pl.ds(r0, WROWS)], out_hbm.at[c].at[pl.ds(r0, WROWS)])


@functools.partial(
    pl.kernel,
    mesh=_mesh,
    out_type=jax.ShapeDtypeStruct((2, 6, N_PAD, 32), jnp.float32),
    scratch_types=[
        pltpu.VMEM((ZROWS, 32), jnp.float32),
        pltpu.VMEM((NBATCH, EBATCH), jnp.int32),
        pltpu.VMEM((NBATCH, EBATCH), jnp.int32),
        pltpu.VMEM((EBATCH, 32), jnp.float32),
        pltpu.VMEM((EBATCH, 32), jnp.float32),
        pltpu.SemaphoreType.DMA,
        pltpu.SemaphoreType.DMA,
        pltpu.VMEM_SHARED((N_PAD, 32), jnp.float32),
    ],
    compiler_params=pltpu.CompilerParams(use_tc_tiling_on_sc=False),
)
def _sc_segh(x_hbm, src_hbm, dst_hbm, zb_hbm, out_hbm,
             zbuf, idxs_all, idxd_all, rows_a, rows_b, sem_a, sem_b, acc):
    c = lax.axis_index("c")
    s = lax.axis_index("s")
    w = c * 16 + s
    pltpu.sync_copy(zb_hbm, zbuf)
    pltpu.sync_copy(src_hbm.at[pl.ds(w * NBATCH, NBATCH)], idxs_all)
    pltpu.sync_copy(dst_hbm.at[pl.ds(w * NBATCH, NBATCH)], idxd_all)
    for j in range(28):
        pltpu.sync_copy(zbuf, acc.at[pl.ds(s * ROWS_PER_TILE + j * ZROWS, ZROWS)])
    plsc.subcore_barrier()
    for k in range(6):
        _seg_pass(x_hbm.at[k], acc, idxs_all, idxd_all,
                  rows_a, rows_b, sem_a, sem_b)
        plsc.subcore_barrier()
        for j in range(4):
            r0 = s * ROWS_PER_TILE + j * WROWS
            pltpu.async_copy(acc.at[pl.ds(r0, WROWS)],
                             out_hbm.at[c].at[k].at[pl.ds(r0, WROWS)], sem_a)
        for j in range(4):
            r0 = s * ROWS_PER_TILE + j * WROWS
            pltpu.make_async_copy(acc.at[pl.ds(r0, WROWS)],
                                  out_hbm.at[c].at[k].at[pl.ds(r0, WROWS)],
                                  sem_a).wait()
        if k < 5:
            for j in range(28):
                pltpu.sync_copy(zbuf,
                                acc.at[pl.ds(s * ROWS_PER_TILE + j * ZROWS, ZROWS)])
            plsc.subcore_barrier()


def _tc_prep_body(loc_ref, vel_ref, out_ref):
    l = loc_ref[...]
    v = vel_ref[...]
    rows = l.shape[0]
    g = l.reshape(rows // NPART, NPART, 3)
    canon = (g - jnp.mean(g, axis=1, keepdims=True)).reshape(rows, 3)
    ones = jnp.ones((rows, 1), jnp.float32)
    zeros = jnp.zeros((rows, 9), jnp.float32)
    out_ref[...] = jnp.concatenate(
        [canon[:, 0:1], v[:, 0:1], canon[:, 1:2], v[:, 1:2],
         canon[:, 2:3], v[:, 2:3], ones, zeros], axis=1)


def _tc_prep(loc, vel):
    bb = 2000
    grid = (N // bb,)
    return pl.pallas_call(
        _tc_prep_body,
        grid=grid,
        in_specs=[pl.BlockSpec((bb, 3), lambda i: (i, 0)),
                  pl.BlockSpec((bb, 3), lambda i: (i, 0))],
        out_specs=pl.BlockSpec((bb, 16), lambda i: (i, 0)),
        out_shape=jax.ShapeDtypeStruct((N, 16), jnp.float32),
    )(loc, vel)


def _tc_layer0_body(x_ref, ps_ref, idw_ref, idb_ref, plw_ref, plb_ref,
                    dirw_ref, y_ref):
    x = x_ref[...]
    ps = ps_ref[0] + ps_ref[1]
    cnt = jnp.maximum(ps[:, 6:7], 1.0)
    idW = idw_ref[...]
    plW = plw_ref[...]
    dirW = dirw_ref[...]
    idb = idb_ref[...]
    plb = plb_ref[...]
    pre = []
    for k in range(3):
        xk = x[:, 2 * k:2 * k + 2]
        pk = ps[:, 2 * k:2 * k + 2] / cnt
        ik = jnp.dot(xk, idW.T, preferred_element_type=jnp.float32) + idb
        pl_k = jnp.dot(pk, plW.T, preferred_element_type=jnp.float32) + plb
        pre.append(ik + pl_k)
    d = [jnp.dot(pre[k], dirW.T, preferred_element_type=jnp.float32)
         for k in range(3)]
    dot = sum(pre[k] * d[k] for k in range(3))
    d2 = sum(d[k] * d[k] for k in range(3))
    coef = jnp.where(dot >= 0.0, 0.0, dot / (d2 + EPS))
    for k in range(3):
        yk = pre[k] - coef * d[k]
        y_ref[2 * k] = yk[:, 0:32]
        y_ref[2 * k + 1] = yk[:, 32:64]


def _tc_layer0(x0, ps0, idW, idb, plW, plb, dirW):
    bn = 2000
    grid = (N // bn,)
    return pl.pallas_call(
        _tc_layer0_body,
        grid=grid,
        in_specs=[
            pl.BlockSpec((bn, 16), lambda i: (i, 0)),
            pl.BlockSpec((2, bn, 16), lambda i: (0, i, 0)),
            pl.BlockSpec((HID, 2), lambda i: (0, 0)),
            pl.BlockSpec((1, HID), lambda i: (0, 0)),
            pl.BlockSpec((HID, 2), lambda i: (0, 0)),
            pl.BlockSpec((1, HID), lambda i: (0, 0)),
            pl.BlockSpec((HID, HID), lambda i: (0, 0)),
        ],
        out_specs=pl.BlockSpec((6, bn, 32), lambda i: (0, i, 0)),
        out_shape=jax.ShapeDtypeStruct((6, N, 32), jnp.float32),
    )(x0, ps0, idW, idb, plW, plb, dirW)


def _tc_layerh_body(x_ref, ps_ref, c_ref, idw_ref, idb_ref, plw_ref,
                    plb_ref, dirw_ref, y_ref):
    cnt0 = c_ref[0] + c_ref[1]
    cnt = jnp.maximum(cnt0[:, 6:7], 1.0)
    idW = idw_ref[...]
    plW = plw_ref[...]
    dirW = dirw_ref[...]
    idb = idb_ref[...]
    plb = plb_ref[...]
    xs = []
    pre = []
    for k in range(3):
        xk = jnp.concatenate([x_ref[2 * k], x_ref[2 * k + 1]], axis=1)
        sk = jnp.concatenate([ps_ref[0, 2 * k] + ps_ref[1, 2 * k],
                              ps_ref[0, 2 * k + 1] + ps_ref[1, 2 * k + 1]],
                             axis=1)
        pk = sk / cnt
        ik = jnp.dot(xk, idW.T, preferred_element_type=jnp.float32) + idb
        pl_k = jnp.dot(pk, plW.T, preferred_element_type=jnp.float32) + plb
        xs.append(xk)
        pre.append(ik + pl_k)
    d = [jnp.dot(pre[k], dirW.T, preferred_element_type=jnp.float32)
         for k in range(3)]
    dot = sum(pre[k] * d[k] for k in range(3))
    d2 = sum(d[k] * d[k] for k in range(3))
    coef = jnp.where(dot >= 0.0, 0.0, dot / (d2 + EPS))
    for k in range(3):
        yk = pre[k] - coef * d[k] + xs[k]
        y_ref[2 * k] = yk[:, 0:32]
        y_ref[2 * k + 1] = yk[:, 32:64]


def _tc_layerh(x, ps, ps0, idW, idb, plW, plb, dirW):
    bn = 1000
    grid = (N // bn,)
    return pl.pallas_call(
        _tc_layerh_body,
        grid=grid,
        in_specs=[
            pl.BlockSpec((6, bn, 32), lambda i: (0, i, 0)),
            pl.BlockSpec((2, 6, bn, 32), lambda i: (0, 0, i, 0)),
            pl.BlockSpec((2, bn, 16), lambda i: (0, i, 0)),
            pl.BlockSpec((HID, HID), lambda i: (0, 0)),
            pl.BlockSpec((1, HID), lambda i: (0, 0)),
            pl.BlockSpec((HID, HID), lambda i: (0, 0)),
            pl.BlockSpec((1, HID), lambda i: (0, 0)),
            pl.BlockSpec((HID, HID), lambda i: (0, 0)),
        ],
        out_specs=pl.BlockSpec((6, bn, 32), lambda i: (0, i, 0)),
        out_shape=jax.ShapeDtypeStruct((6, N, 32), jnp.float32),
    )(x, ps, ps0, idW, idb, plW, plb, dirW)


def _tc_final_body(x_ref, ow_ref, ob_ref, out_ref):
    oW = ow_ref[...]
    ob = ob_ref[...]
    outs = []
    for k in range(3):
        xk = jnp.concatenate([x_ref[2 * k], x_ref[2 * k + 1]], axis=1)
        rows = xk.shape[0]
        g = jnp.mean(xk.reshape(rows // NPART, NPART, HID), axis=1)
        outs.append(jnp.dot(g, oW.T, preferred_element_type=jnp.float32) + ob)
    out_ref[...] = jnp.stack(outs, axis=1)


def _tc_final(x, oW, ob):
    bb = 400
    grid = (B // bb,)
    return pl.pallas_call(
        _tc_final_body,
        grid=grid,
        in_specs=[
            pl.BlockSpec((6, bb * NPART, 32), lambda i: (0, i, 0)),
            pl.BlockSpec((4, HID), lambda i: (0, 0)),
            pl.BlockSpec((1, 4), lambda i: (0, 0)),
        ],
        out_specs=pl.BlockSpec((bb, 3, 4), lambda i: (i, 0, 0)),
        out_shape=jax.ShapeDtypeStruct((B, 3, 4), jnp.float32),
    )(x, oW, ob)


def kernel(nodes, loc, edges, vel, edge_attr, charges, params):
    src = edges[0]
    dst = edges[1]
    srcp = jnp.concatenate([src, jnp.zeros((E_PAD - E,), jnp.int32)])
    dstp = jnp.concatenate([dst, jnp.full((E_PAD - E,), N, jnp.int32)])
    srcp = srcp.reshape(NW * NBATCH, EBATCH)
    dstp = dstp.reshape(NW * NBATCH, EBATCH)
    zb16 = jnp.zeros((ZROWS, 16), jnp.float32)
    zb32 = jnp.zeros((ZROWS, 32), jnp.float32)

    x0 = _tc_prep(loc, vel)
    ps0 = _sc_seg0(x0, srcp, dstp, zb16)
    x = _tc_layer0(x0, ps0,
                   params["id_W0"], params["id_b0"].reshape(1, HID),
                   params["pool_W0"], params["pool_b0"].reshape(1, HID),
                   params["dir_W0"])
    for i in range(1, 4):
        ps = _sc_segh(x, srcp, dstp, zb32)
        x = _tc_layerh(x, ps, ps0,
                       params["id_W%d" % i], params["id_b%d" % i].reshape(1, HID),
                       params["pool_W%d" % i], params["pool_b%d" % i].reshape(1, HID),
                       params["dir_W%d" % i])
    o = _tc_final(x, params["out_W"], params["out_b"].reshape(1, 4))
    o = jnp.swapaxes(o, 1, 2)
    return o[:, :3, :], o[:, 3:, :]


# speedup vs baseline: 34.5910x; 1.0021x over previous
import functools

import jax
import jax.numpy as jnp
from jax import lax
from jax.experimental import pallas as pl
from jax.experimental.pallas import tpu as pltpu
from jax.experimental.pallas import tpu_sc as plsc

N = 50000
E = 200000
B = 10000
NPART = 5
HID = 64
EPS = 1e-6

NW = 32
N_PAD = 50176
ROWS_PER_TILE = N_PAD // 16
WROWS = 784
ZROWS = 112
E_PAD = 200704
EDGES_PER_W = E_PAD // NW
EBATCH = 128
NBATCH = EDGES_PER_W // EBATCH

_mesh = plsc.VectorSubcoreMesh(core_axis_name="c", subcore_axis_name="s")


def _seg_pass(xk, acc, idxs_all, idxd_all, rows_a, rows_b, sem_a, sem_b):
    pltpu.async_copy(xk.at[idxs_all.at[0]], rows_a, sem_a)

    def body(i, carry):
        b0 = 2 * i
        pltpu.async_copy(xk.at[idxs_all.at[b0 + 1]], rows_b, sem_b)
        pltpu.make_async_copy(xk.at[idxs_all.at[b0]], rows_a, sem_a).wait()
        pltpu.sync_copy(rows_a, acc.at[idxd_all.at[b0]], add=True)
        pltpu.async_copy(xk.at[idxs_all.at[b0 + 2]], rows_a, sem_a)
        pltpu.make_async_copy(xk.at[idxs_all.at[b0 + 1]], rows_b, sem_b).wait()
        pltpu.sync_copy(rows_b, acc.at[idxd_all.at[b0 + 1]], add=True)
        return carry

    lax.fori_loop(0, (NBATCH - 1) // 2, body, 0)
    pltpu.make_async_copy(xk.at[idxs_all.at[NBATCH - 1]], rows_a, sem_a).wait()
    pltpu.sync_copy(rows_a, acc.at[idxd_all.at[NBATCH - 1]], add=True)


@functools.partial(
    pl.kernel,
    mesh=_mesh,
    out_type=jax.ShapeDtypeStruct((2, N_PAD, 16), jnp.float32),
    scratch_types=[
        pltpu.VMEM((ZROWS, 16), jnp.float32),
        pltpu.VMEM((NBATCH, EBATCH), jnp.int32),
        pltpu.VMEM((NBATCH, EBATCH), jnp.int32),
        pltpu.VMEM((EBATCH, 16), jnp.float32),
        pltpu.VMEM((EBATCH, 16), jnp.float32),
        pltpu.SemaphoreType.DMA,
        pltpu.SemaphoreType.DMA,
        pltpu.VMEM_SHARED((N_PAD, 16), jnp.float32),
    ],
    compiler_params=pltpu.CompilerParams(use_tc_tiling_on_sc=False),
)
def _sc_seg0(x_hbm, src_hbm, dst_hbm, zb_hbm, out_hbm,
             zbuf, idxs_all, idxd_all, rows_a, rows_b, sem_a, sem_b, acc):
    c = lax.axis_index("c")
    s = lax.axis_index("s")
    w = c * 16 + s
    pltpu.sync_copy(zb_hbm, zbuf)
    pltpu.sync_copy(src_hbm.at[pl.ds(w * NBATCH, NBATCH)], idxs_all)
    pltpu.sync_copy(dst_hbm.at[pl.ds(w * NBATCH, NBATCH)], idxd_all)
    for j in range(28):
        pltpu.sync_copy(zbuf, acc.at[pl.ds(s * ROWS_PER_TILE + j * ZROWS, ZROWS)])
    plsc.subcore_barrier()
    _seg_pass(x_hbm, acc, idxs_all, idxd_all, rows_a, rows_b, sem_a, sem_b)
    plsc.subcore_barrier()
    for j in range(4):
        r0 = s * ROWS_PER_TILE + j * WROWS
        pltpu.sync_copy(acc.at[pl.ds(r0, WROWS)], out_hbm.at[c].at[pl.ds(r0, WROWS)])


@functools.partial(
    pl.kernel,
    mesh=_mesh,
    out_type=jax.ShapeDtypeStruct((2, 6, N_PAD, 32), jnp.float32),
    scratch_types=[
        pltpu.VMEM((ZROWS, 32), jnp.float32),
        pltpu.VMEM((NBATCH, EBATCH), jnp.int32),
        pltpu.VMEM((NBATCH, EBATCH), jnp.int32),
        pltpu.VMEM((EBATCH, 32), jnp.float32),
        pltpu.VMEM((EBATCH, 32), jnp.float32),
        pltpu.SemaphoreType.DMA,
        pltpu.SemaphoreType.DMA,
        pltpu.VMEM_SHARED((N_PAD, 32), jnp.float32),
    ],
    compiler_params=pltpu.CompilerParams(use_tc_tiling_on_sc=False),
)
def _sc_segh(x_hbm, src_hbm, dst_hbm, zb_hbm, out_hbm,
             zbuf, idxs_all, idxd_all, rows_a, rows_b, sem_a, sem_b, acc):
    c = lax.axis_index("c")
    s = lax.axis_index("s")
    w = c * 16 + s
    pltpu.sync_copy(zb_hbm, zbuf)
    pltpu.sync_copy(src_hbm.at[pl.ds(w * NBATCH, NBATCH)], idxs_all)
    pltpu.sync_copy(dst_hbm.at[pl.ds(w * NBATCH, NBATCH)], idxd_all)
    for j in range(28):
        pltpu.sync_copy(zbuf, acc.at[pl.ds(s * ROWS_PER_TILE + j * ZROWS, ZROWS)])
    plsc.subcore_barrier()
    for k in range(6):
        _seg_pass(x_hbm.at[k], acc, idxs_all, idxd_all,
                  rows_a, rows_b, sem_a, sem_b)
        plsc.subcore_barrier()
        for j in range(4):
            r0 = s * ROWS_PER_TILE + j * WROWS
            pltpu.async_copy(acc.at[pl.ds(r0, WROWS)],
                             out_hbm.at[c].at[k].at[pl.ds(r0, WROWS)], sem_a)
        for j in range(4):
            r0 = s * ROWS_PER_TILE + j * WROWS
            pltpu.make_async_copy(acc.at[pl.ds(r0, WROWS)],
                                  out_hbm.at[c].at[k].at[pl.ds(r0, WROWS)],
                                  sem_a).wait()
        if k < 5:
            for j in range(28):
                pltpu.sync_copy(zbuf,
                                acc.at[pl.ds(s * ROWS_PER_TILE + j * ZROWS, ZROWS)])
            plsc.subcore_barrier()


def _tc_prep_body(loc_ref, vel_ref, out_ref):
    l = loc_ref[...]
    v = vel_ref[...]
    rows = l.shape[0]
    g = l.reshape(rows // NPART, NPART, 3)
    canon = (g - jnp.mean(g, axis=1, keepdims=True)).reshape(rows, 3)
    ones = jnp.ones((rows, 1), jnp.float32)
    zeros = jnp.zeros((rows, 9), jnp.float32)
    out_ref[...] = jnp.concatenate(
        [canon[:, 0:1], v[:, 0:1], canon[:, 1:2], v[:, 1:2],
         canon[:, 2:3], v[:, 2:3], ones, zeros], axis=1)


def _tc_prep(loc, vel):
    bb = 2000
    grid = (N // bb,)
    return pl.pallas_call(
        _tc_prep_body,
        grid=grid,
        in_specs=[pl.BlockSpec((bb, 3), lambda i: (i, 0)),
                  pl.BlockSpec((bb, 3), lambda i: (i, 0))],
        out_specs=pl.BlockSpec((bb, 16), lambda i: (i, 0)),
        out_shape=jax.ShapeDtypeStruct((N, 16), jnp.float32),
    )(loc, vel)


def _tc_layer0_body(x_ref, ps_ref, idw_ref, idb_ref, plw_ref, plb_ref,
                    dirw_ref, y_ref):
    x = x_ref[...]
    ps = ps_ref[0] + ps_ref[1]
    cnt = jnp.maximum(ps[:, 6:7], 1.0)
    idW = idw_ref[...]
    plW = plw_ref[...]
    dirW = dirw_ref[...]
    idb = idb_ref[...]
    plb = plb_ref[...]
    pre = []
    for k in range(3):
        xk = x[:, 2 * k:2 * k + 2]
        pk = ps[:, 2 * k:2 * k + 2] / cnt
        ik = jnp.dot(xk, idW.T, preferred_element_type=jnp.float32) + idb
        pl_k = jnp.dot(pk, plW.T, preferred_element_type=jnp.float32) + plb
        pre.append(ik + pl_k)
    d = [jnp.dot(pre[k], dirW.T, preferred_element_type=jnp.float32)
         for k in range(3)]
    dot = sum(pre[k] * d[k] for k in range(3))
    d2 = sum(d[k] * d[k] for k in range(3))
    coef = jnp.where(dot >= 0.0, 0.0, dot / (d2 + EPS))
    for k in range(3):
        yk = pre[k] - coef * d[k]
        y_ref[2 * k] = yk[:, 0:32]
        y_ref[2 * k + 1] = yk[:, 32:64]


def _tc_layer0(x0, ps0, idW, idb, plW, plb, dirW):
    bn = 2000
    grid = (N // bn,)
    return pl.pallas_call(
        _tc_layer0_body,
        grid=grid,
        in_specs=[
            pl.BlockSpec((bn, 16), lambda i: (i, 0)),
            pl.BlockSpec((2, bn, 16), lambda i: (0, i, 0)),
            pl.BlockSpec((HID, 2), lambda i: (0, 0)),
            pl.BlockSpec((1, HID), lambda i: (0, 0)),
            pl.BlockSpec((HID, 2), lambda i: (0, 0)),
            pl.BlockSpec((1, HID), lambda i: (0, 0)),
            pl.BlockSpec((HID, HID), lambda i: (0, 0)),
        ],
        out_specs=pl.BlockSpec((6, bn, 32), lambda i: (0, i, 0)),
        out_shape=jax.ShapeDtypeStruct((6, N, 32), jnp.float32),
    )(x0, ps0, idW, idb, plW, plb, dirW)


def _tc_layerh_body(x_ref, ps_ref, c_ref, idw_ref, idb_ref, plw_ref,
                    plb_ref, dirw_ref, y_ref):
    cnt0 = c_ref[0] + c_ref[1]
    cnt = jnp.maximum(cnt0[:, 6:7], 1.0)
    idW = idw_ref[...]
    plW = plw_ref[...]
    dirW = dirw_ref[...]
    idb = idb_ref[...]
    plb = plb_ref[...]
    xs = []
    pre = []
    for k in range(3):
        xk = jnp.concatenate([x_ref[2 * k], x_ref[2 * k + 1]], axis=1)
        sk = jnp.concatenate([ps_ref[0, 2 * k] + ps_ref[1, 2 * k],
                              ps_ref[0, 2 * k + 1] + ps_ref[1, 2 * k + 1]],
                             axis=1)
        pk = sk / cnt
        ik = jnp.dot(xk, idW.T, preferred_element_type=jnp.float32) + idb
        pl_k = jnp.dot(pk, plW.T, preferred_element_type=jnp.float32) + plb
        xs.append(xk)
        pre.append(ik + pl_k)
    d = [jnp.dot(pre[k], dirW.T, preferred_element_type=jnp.float32)
         for k in range(3)]
    dot = sum(pre[k] * d[k] for k in range(3))
    d2 = sum(d[k] * d[k] for k in range(3))
    coef = jnp.where(dot >= 0.0, 0.0, dot / (d2 + EPS))
    for k in range(3):
        yk = pre[k] - coef * d[k] + xs[k]
        y_ref[2 * k] = yk[:, 0:32]
        y_ref[2 * k + 1] = yk[:, 32:64]


def _tc_layerh(x, ps, ps0, idW, idb, plW, plb, dirW):
    bn = 2000
    grid = (N // bn,)
    return pl.pallas_call(
        _tc_layerh_body,
        compiler_params=pltpu.CompilerParams(vmem_limit_bytes=134217728),
        grid=grid,
        in_specs=[
            pl.BlockSpec((6, bn, 32), lambda i: (0, i, 0)),
            pl.BlockSpec((2, 6, bn, 32), lambda i: (0, 0, i, 0)),
            pl.BlockSpec((2, bn, 16), lambda i: (0, i, 0)),
            pl.BlockSpec((HID, HID), lambda i: (0, 0)),
            pl.BlockSpec((1, HID), lambda i: (0, 0)),
            pl.BlockSpec((HID, HID), lambda i: (0, 0)),
            pl.BlockSpec((1, HID), lambda i: (0, 0)),
            pl.BlockSpec((HID, HID), lambda i: (0, 0)),
        ],
        out_specs=pl.BlockSpec((6, bn, 32), lambda i: (0, i, 0)),
        out_shape=jax.ShapeDtypeStruct((6, N, 32), jnp.float32),
    )(x, ps, ps0, idW, idb, plW, plb, dirW)


def _tc_final_body(x_ref, ow_ref, ob_ref, out_ref):
    oW = ow_ref[...]
    ob = ob_ref[...]
    outs = []
    for k in range(3):
        xk = jnp.concatenate([x_ref[2 * k], x_ref[2 * k + 1]], axis=1)
        rows = xk.shape[0]
        g = jnp.mean(xk.reshape(rows // NPART, NPART, HID), axis=1)
        outs.append(jnp.dot(g, oW.T, preferred_element_type=jnp.float32) + ob)
    out_ref[...] = jnp.stack(outs, axis=1)


def _tc_final(x, oW, ob):
    bb = 400
    grid = (B // bb,)
    return pl.pallas_call(
        _tc_final_body,
        grid=grid,
        in_specs=[
            pl.BlockSpec((6, bb * NPART, 32), lambda i: (0, i, 0)),
            pl.BlockSpec((4, HID), lambda i: (0, 0)),
            pl.BlockSpec((1, 4), lambda i: (0, 0)),
        ],
        out_specs=pl.BlockSpec((bb, 3, 4), lambda i: (i, 0, 0)),
        out_shape=jax.ShapeDtypeStruct((B, 3, 4), jnp.float32),
    )(x, oW, ob)


def kernel(nodes, loc, edges, vel, edge_attr, charges, params):
    src = edges[0]
    dst = edges[1]
    srcp = jnp.concatenate([src, jnp.zeros((E_PAD - E,), jnp.int32)])
    dstp = jnp.concatenate([dst, jnp.full((E_PAD - E,), N, jnp.int32)])
    srcp = srcp.reshape(NW * NBATCH, EBATCH)
    dstp = dstp.reshape(NW * NBATCH, EBATCH)
    zb16 = jnp.zeros((ZROWS, 16), jnp.float32)
    zb32 = jnp.zeros((ZROWS, 32), jnp.float32)

    x0 = _tc_prep(loc, vel)
    ps0 = _sc_seg0(x0, srcp, dstp, zb16)
    x = _tc_layer0(x0, ps0,
                   params["id_W0"], params["id_b0"].reshape(1, HID),
                   params["pool_W0"], params["pool_b0"].reshape(1, HID),
                   params["dir_W0"])
    for i in range(1, 4):
        ps = _sc_segh(x, srcp, dstp, zb32)
        x = _tc_layerh(x, ps, ps0,
                       params["id_W%d" % i], params["id_b%d" % i].reshape(1, HID),
                       params["pool_W%d" % i], params["pool_b%d" % i].reshape(1, HID),
                       params["dir_W%d" % i])
    o = _tc_final(x, params["out_W"], params["out_b"].reshape(1, 4))
    o = jnp.swapaxes(o, 1, 2)
    return o[:, :3, :], o[:, 3:, :]
